# baseline (speedup 1.0000x reference)
import functools

import jax
import jax.numpy as jnp
from jax import lax
from jax.experimental import pallas as pl
from jax.experimental.pallas import tpu as pltpu
from jax.experimental.pallas import tpu_sc as plsc

N = 10000
E = 320000
D = 64
F = 128
BSZ = 4096
NC, NS = 2, 16
EC = 128
CPT = 160
EPAD = NS * CPT * EC
NPAD = 10240
RPT = NPAD // NS
IC = BSZ // (NS * 128)

_mesh = plsc.VectorSubcoreMesh(core_axis_name="c", subcore_axis_name="s")


def _deg_body(d0, d1, d2, d3, zfill, ones_hbm,
              deg0, deg1, deg2, deg3,
              acc0, acc1, onesb, zb, dstb, tmpb, dsem):
    core = lax.axis_index("c")
    sid = lax.axis_index("s")
    edges = (d0, d1, d2, d3)
    degs = (deg0, deg1, deg2, deg3)
    accs = (acc0, acc1)

    pltpu.sync_copy(ones_hbm, onesb)
    pltpu.sync_copy(zfill, zb)
    for a in accs:
        pltpu.sync_copy(zb, a.at[pl.ds(sid * RPT, RPT)])

    plsc.subcore_barrier()

    for g in range(4):
        @pl.when(core == g // 2)
        def _(g=g):
            a = accs[g % 2]
            pltpu.sync_copy(edges[g].at[1, pl.ds(sid * CPT, CPT)], dstb)

            def body(p, carry):
                j = p * 8
                for k in range(8):
                    pltpu.async_copy(
                        onesb, a.at[dstb.at[j + k]], dsem, add=True)
                for k in range(8):
                    pltpu.make_async_copy(
                        onesb, a.at[dstb.at[j + k]], dsem).wait()
                return carry

            lax.fori_loop(0, CPT // 8, body, 0)

    plsc.subcore_barrier()

    for g in range(4):
        @pl.when((core == g // 2) & (sid < 10))
        def _(g=g):
            a = accs[g % 2]
            pltpu.sync_copy(a.at[pl.ds(sid * 1000, 1000)], tmpb)
            pltpu.sync_copy(tmpb, degs[g].at[pl.ds(sid * 1000, 1000)])


_sc_params = pltpu.CompilerParams(use_tc_tiling_on_sc=False,
                                  needs_layout_passes=False)

_deg_call = pl.kernel(
    _deg_body,
    out_type=[jax.ShapeDtypeStruct((N,), jnp.float32)] * 4,
    mesh=_mesh,
    compiler_params=_sc_params,
    scratch_types=[
        pltpu.VMEM_SHARED((NPAD,), jnp.float32),
        pltpu.VMEM_SHARED((NPAD,), jnp.float32),
        pltpu.VMEM((EC,), jnp.float32),
        pltpu.VMEM((RPT,), jnp.float32),
        pltpu.VMEM((CPT, EC), jnp.int32),
        pltpu.VMEM((1000,), jnp.float32),
        pltpu.SemaphoreType.DMA,
    ],
)


def _z_body(x0, x1, x2, x3, w0, w1, w2, w3, dg0, dg1, dg2, dg3,
            z0, z1, z2, z3):
    for x_ref, w_ref, deg_ref, z_ref in (
            (x0, w0, dg0, z0), (x1, w1, dg1, z1),
            (x2, w2, dg2, z2), (x3, w3, dg3, z3)):
        nrm = lax.rsqrt(deg_ref[...] + 1.0)
        y = jnp.dot(x_ref[...], w_ref[...],
                    preferred_element_type=jnp.float32,
                    precision=lax.Precision.HIGHEST)
        z_ref[...] = y * nrm


_z_call = pl.pallas_call(
    _z_body,
    grid=(10,),
    in_specs=([pl.BlockSpec((1000, F), lambda b: (b, 0))] * 4
              + [pl.BlockSpec((F, D), lambda b: (0, 0))] * 4
              + [pl.BlockSpec((1000, 1), lambda b: (b, 0))] * 4),
    out_specs=[pl.BlockSpec((1000, D), lambda b: (b, 0))] * 4,
    out_shape=[jax.ShapeDtypeStruct((N, D), jnp.float32)] * 4,
)


def _mp_body(e0, e1, e2, e3,
             z0, z1, z2, z3, n0, n1, n2, n3,
             u_r, i_r, attn_a, attn_b, zf2,
             gu0, gu1, gu2, gu3, gi0, gi1, gi2, gi3, oaa, obb,
             acc, srcb, dstb, rows, rows2, rows3, rows4,
             idxb, grows, zrows, nbuf, obuf, abuf,
             gs0, gs1, gs2, gs3, ss0, ss1, ss2, ss3):
    core = lax.axis_index("c")
    sid = lax.axis_index("s")
    bufs = (rows, rows2, rows3, rows4)
    gsems = (gs0, gs1, gs2, gs3)
    ssems = (ss0, ss1, ss2, ss3)
    sem = gs0
    edges = (e0, e1, e2, e3)
    zs = (z0, z1, z2, z3)
    ns = (n0, n1, n2, n3)
    gus = (gu0, gu1, gu2, gu3)
    gis = (gi0, gi1, gi2, gi3)

    for gi in range(2):
        pltpu.sync_copy(zf2, rows)

        def zbody(p, carry):
            pltpu.sync_copy(rows, acc.at[pl.ds(sid * RPT + p * EC, EC)])
            return carry

        lax.fori_loop(0, RPT // EC, zbody, 0)
        plsc.subcore_barrier()

        for g in (gi, 2 + gi):
            @pl.when(core == g // 2)
            def _(g=g):
                def slab(t, carry):
                    base = sid * CPT + t * 16
                    pltpu.sync_copy(edges[g].at[0, pl.ds(base, 16)], srcb)
                    pltpu.sync_copy(edges[g].at[1, pl.ds(base, 16)], dstb)
                    for k in range(4):
                        pltpu.async_copy(
                            zs[g].at[srcb.at[k]], bufs[k], gsems[k])

                    def body(p, c2):
                        j = p * 4
                        for k in range(4):
                            pltpu.make_async_copy(
                                zs[g].at[srcb.at[j + k]], bufs[k],
                                gsems[k]).wait()
                            pltpu.async_copy(
                                bufs[k], acc.at[dstb.at[j + k]],
                                ssems[k], add=True)
                            if k >= 2:
                                _refill(p, j, k - 2)
                        _refill(p, j, 2)
                        _refill(p, j, 3)
                        return c2

                    def _refill(p, j, k):
                        pltpu.make_async_copy(
                            bufs[k], acc.at[dstb.at[j + k]], ssems[k]).wait()

                        @pl.when(p < 3)
                        def _():
                            pltpu.async_copy(
                                zs[g].at[srcb.at[j + 4 + k]], bufs[k],
                                gsems[k])

                    lax.fori_loop(0, 4, body, 0)
                    return carry

                lax.fori_loop(0, CPT // 16, slab, 0)

        plsc.subcore_barrier()

        for g in (gi, 2 + gi):
            @pl.when(core == g // 2)
            def _(g=g):
                for idx_src, out_ref in ((u_r, gus[g]), (i_r, gis[g])):
                    def rbody(kk, carry, idx_src=idx_src, out_ref=out_ref):
                        pltpu.sync_copy(idx_src.at[sid * IC + kk], idxb)
                        pltpu.async_copy(acc.at[idxb], grows, gs1)
                        pltpu.async_copy(zs[g].at[idxb], zrows, gs2)
                        pltpu.async_copy(ns[g].at[idxb], nbuf, gs3)
                        pltpu.make_async_copy(ns[g].at[idxb], nbuf, gs3).wait()

                        def rsq(q, c2):
                            sl = pl.ds(q * 16, 16)
                            v = nbuf[sl] + 1.0
                            iv = plsc.bitcast(v, jnp.int32)
                            iv = 0x5F3759DF - lax.shift_right_logical(iv, 1)
                            y = plsc.bitcast(iv, jnp.float32)
                            for _ in range(3):
                                y = y * (1.5 - 0.5 * v * y * y)
                            nbuf[sl] = y
                            return c2

                        lax.fori_loop(0, 8, rsq, 0)
                        pltpu.make_async_copy(acc.at[idxb], grows, gs1).wait()
                        pltpu.make_async_copy(
                            zs[g].at[idxb], zrows, gs2).wait()

                        def comp(q, c2):
                            nv16 = nbuf[pl.ds(q * 16, 16)]
                            for rr in range(16):
                                r = q * 16 + rr
                                nv = nv16[rr]
                                for c in range(D // 16):
                                    sl = pl.ds(c * 16, 16)
                                    v = (grows[r, sl] + zrows[r, sl]) * nv
                                    obuf[r, sl] = jnp.maximum(v, 0.0)
                            return c2

                        lax.fori_loop(0, 8, comp, 0)
                        pltpu.sync_copy(
                            obuf,
                            out_ref.at[pl.ds((sid * IC + kk) * 128, 128)])
                        return carry

                    lax.fori_loop(0, IC, rbody, 0)

        plsc.subcore_barrier()

    for cid, tab, out_ref in ((0, attn_a, oaa), (1, attn_b, obb)):
        @pl.when(core == cid)
        def _(tab=tab, out_ref=out_ref):
            def ab(kk, carry):
                pltpu.sync_copy(u_r.at[sid * IC + kk], idxb)
                pltpu.async_copy(tab.at[idxb], abuf, sem).wait()
                pltpu.sync_copy(
                    abuf, out_ref.at[pl.ds((sid * IC + kk) * 128, 128)])
                return carry

            lax.fori_loop(0, IC, ab, 0)


_mp_call = pl.kernel(
    _mp_body,
    out_type=([jax.ShapeDtypeStruct((BSZ, D), jnp.float32)] * 8
              + [jax.ShapeDtypeStruct((BSZ, F), jnp.float32)] * 2),
    mesh=_mesh,
    compiler_params=_sc_params,
    scratch_types=[
        pltpu.VMEM_SHARED((NPAD, D), jnp.float32),
        pltpu.VMEM((16, EC), jnp.int32),
        pltpu.VMEM((16, EC), jnp.int32),
        pltpu.VMEM((EC, D), jnp.float32),
        pltpu.VMEM((EC, D), jnp.float32),
        pltpu.VMEM((EC, D), jnp.float32),
        pltpu.VMEM((EC, D), jnp.float32),
        pltpu.VMEM((128,), jnp.int32),
        pltpu.VMEM((128, D), jnp.float32),
        pltpu.VMEM((128, D), jnp.float32),
        pltpu.VMEM((128,), jnp.float32),
        pltpu.VMEM((128, D), jnp.float32),
        pltpu.VMEM((128, F), jnp.float32),
    ] + [pltpu.SemaphoreType.DMA] * 8,
)


def _head_body(dom, g0u, g1u, g2u, g3u, g0i, g1i, g2i, g3i, waa, wbb,
               uw1, ub1, uw2, ub2, iw1, ib1, iw2, ib2,
               user_ref, item_ref):
    is_a = dom[0] == 0
    u_a = jnp.concatenate([g0u[...], g1u[...]], axis=1)
    u_b = jnp.concatenate([g2u[...], g3u[...]], axis=1)
    i_a = jnp.concatenate([g0i[...], g1i[...]], axis=1)
    i_b = jnp.concatenate([g2i[...], g3i[...]], axis=1)
    w_aa = waa[...]
    w_bb = wbb[...]
    u_from_a = u_a * w_aa + u_b * (1.0 - w_aa)
    u_from_b = u_b * w_bb + u_a * (1.0 - w_bb)
    user_in = jnp.where(is_a, u_from_a, u_from_b)
    item_in = jnp.where(is_a, i_a, i_b)

    def mlp(x, w1, b1, w2, b2):
        h = jnp.dot(x, w1[...], preferred_element_type=jnp.float32,
                    precision=lax.Precision.HIGHEST) + b1[...]
        h = jnp.maximum(h, 0.0)
        return jnp.dot(h, w2[...], preferred_element_type=jnp.float32,
                       precision=lax.Precision.HIGHEST) + b2[...]

    user_ref[...] = mlp(user_in, uw1, ub1, uw2, ub2)
    item_ref[...] = mlp(item_in, iw1, ib1, iw2, ib2)


_BLK = 512
_spec_d = pl.BlockSpec((_BLK, D), lambda b: (b, 0))
_spec_f = pl.BlockSpec((_BLK, F), lambda b: (b, 0))
_spec_g = pl.BlockSpec((NS * IC, 128), lambda b: (0, 0))
_spec_w = pl.BlockSpec((F, F), lambda b: (0, 0))
_spec_b = pl.BlockSpec((1, F), lambda b: (0, 0))

_head_call = pl.pallas_call(
    _head_body,
    grid=(BSZ // _BLK,),
    in_specs=([pl.BlockSpec(memory_space=pltpu.SMEM)]
              + [_spec_d] * 8 + [_spec_f] * 2
              + [_spec_w, _spec_b, _spec_w, _spec_b] * 2),
    out_specs=[_spec_f, _spec_f],
    out_shape=[jax.ShapeDtypeStruct((BSZ, F), jnp.float32)] * 2,
)


def kernel(u, i, domain, edge_rating_A, edge_review_A, edge_rating_B,
           edge_review_B, feat_rating_A, feat_review_A, feat_rating_B,
           feat_review_B, W_rat_A, W_rev_A, W_rat_B, W_rev_B, attn_AA,
           attn_BB, umlp_W1, umlp_b1, umlp_W2, umlp_b2, imlp_W1, imlp_b1,
           imlp_W2, imlp_b2):
    f32 = jnp.float32
    edges = (edge_rating_A, edge_review_A, edge_rating_B, edge_review_B)
    feats = (feat_rating_A, feat_review_A, feat_rating_B, feat_review_B)
    ws = (W_rat_A, W_rev_A, W_rat_B, W_rev_B)

    fill = jnp.broadcast_to(jnp.array([[0], [N]], jnp.int32), (2, EPAD - E))
    edges_r = [jnp.concatenate([e.astype(jnp.int32), fill], axis=1)
               .reshape(2, NS * CPT, EC) for e in edges]
    u_r = u.astype(jnp.int32).reshape(NS * IC, 128)
    i_r = i.astype(jnp.int32).reshape(NS * IC, 128)

    zfill = jnp.zeros((RPT,), f32)
    ones_hbm = jnp.ones((EC,), f32)
    zf2 = jnp.zeros((EC, D), f32)

    degs = _deg_call(*edges_r, zfill, ones_hbm)

    zs = _z_call(*[feats[g].astype(f32) for g in range(4)],
                 *[ws[g].astype(f32) for g in range(4)],
                 *[degs[g].reshape(N, 1) for g in range(4)])

    mp_out = _mp_call(*edges_r, *zs, *degs, u_r, i_r,
                      attn_AA.astype(f32), attn_BB.astype(f32), zf2)
    gus = mp_out[0:4]
    gis = mp_out[4:8]
    oaa, obb = mp_out[8], mp_out[9]

    dom = jnp.asarray(domain, jnp.int32).reshape(1)
    user, item = _head_call(
        dom, gus[0], gus[1], gus[2], gus[3], gis[0], gis[1], gis[2], gis[3],
        oaa, obb, umlp_W1, umlp_b1.reshape(1, F), umlp_W2,
        umlp_b2.reshape(1, F), imlp_W1, imlp_b1.reshape(1, F), imlp_W2,
        imlp_b2.reshape(1, F))

    return (user, item, gus[1], gus[3])


# speedup vs baseline: 1.7516x; 1.7516x over previous
import functools

import jax
import jax.numpy as jnp
from jax import lax
from jax.experimental import pallas as pl
from jax.experimental.pallas import tpu as pltpu
from jax.experimental.pallas import tpu_sc as plsc

N = 10000
E = 320000
D = 64
F = 128
BSZ = 4096
NC, NS = 2, 16
EC = 128
CPT = 160
EPAD = NS * CPT * EC
NPAD = 10240
RPT = NPAD // NS
IC = BSZ // (NS * 128)

_mesh = plsc.VectorSubcoreMesh(core_axis_name="c", subcore_axis_name="s")


def _deg_body(d0, d1, d2, d3, zfill, ones_hbm,
              deg0, deg1, deg2, deg3,
              acc0, acc1, onesb, zb, dstb, tmpb, dsem):
    core = lax.axis_index("c")
    sid = lax.axis_index("s")
    edges = (d0, d1, d2, d3)
    degs = (deg0, deg1, deg2, deg3)
    accs = (acc0, acc1)

    pltpu.sync_copy(ones_hbm, onesb)
    pltpu.sync_copy(zfill, zb)
    for a in accs:
        pltpu.sync_copy(zb, a.at[pl.ds(sid * RPT, RPT)])

    plsc.subcore_barrier()

    for g in range(4):
        @pl.when(core == g // 2)
        def _(g=g):
            a = accs[g % 2]
            pltpu.sync_copy(edges[g].at[1, pl.ds(sid * CPT, CPT)], dstb)

            def body(p, carry):
                j = p * 8
                for k in range(8):
                    pltpu.async_copy(
                        onesb, a.at[dstb.at[j + k]], dsem, add=True)
                for k in range(8):
                    pltpu.make_async_copy(
                        onesb, a.at[dstb.at[j + k]], dsem).wait()
                return carry

            lax.fori_loop(0, CPT // 8, body, 0)

    plsc.subcore_barrier()

    for g in range(4):
        @pl.when((core == g // 2) & (sid < 10))
        def _(g=g):
            a = accs[g % 2]
            pltpu.sync_copy(a.at[pl.ds(sid * 1000, 1000)], tmpb)
            pltpu.sync_copy(tmpb, degs[g].at[pl.ds(sid * 1000, 1000)])


_sc_params = pltpu.CompilerParams(use_tc_tiling_on_sc=False,
                                  needs_layout_passes=False)

_deg_call = pl.kernel(
    _deg_body,
    out_type=[jax.ShapeDtypeStruct((N,), jnp.float32)] * 4,
    mesh=_mesh,
    compiler_params=_sc_params,
    scratch_types=[
        pltpu.VMEM_SHARED((NPAD,), jnp.float32),
        pltpu.VMEM_SHARED((NPAD,), jnp.float32),
        pltpu.VMEM((EC,), jnp.float32),
        pltpu.VMEM((RPT,), jnp.float32),
        pltpu.VMEM((CPT, EC), jnp.int32),
        pltpu.VMEM((1000,), jnp.float32),
        pltpu.SemaphoreType.DMA,
    ],
)


def _z_body(x0, x1, x2, x3, w0, w1, w2, w3, dg0, dg1, dg2, dg3,
            z0, z1, z2, z3):
    for x_ref, w_ref, deg_ref, z_ref in (
            (x0, w0, dg0, z0), (x1, w1, dg1, z1),
            (x2, w2, dg2, z2), (x3, w3, dg3, z3)):
        nrm = lax.rsqrt(deg_ref[...] + 1.0)
        y = jnp.dot(x_ref[...], w_ref[...],
                    preferred_element_type=jnp.float32,
                    precision=lax.Precision.HIGHEST)
        z_ref[...] = y * nrm


_z_call = pl.pallas_call(
    _z_body,
    grid=(10,),
    in_specs=([pl.BlockSpec((1000, F), lambda b: (b, 0))] * 4
              + [pl.BlockSpec((F, D), lambda b: (0, 0))] * 4
              + [pl.BlockSpec((1000, 1), lambda b: (b, 0))] * 4),
    out_specs=[pl.BlockSpec((1000, D), lambda b: (b, 0))] * 4,
    out_shape=[jax.ShapeDtypeStruct((N, D), jnp.float32)] * 4,
)


def _mp_body(e0, e1, e2, e3,
             z0, z1, z2, z3, n0, n1, n2, n3,
             u_r, i_r, attn_a, attn_b, zf2,
             gu0, gu1, gu2, gu3, gi0, gi1, gi2, gi3, oaa, obb,
             acc, srcb, dstb, rows, rows2, rows3, rows4,
             idxb, grows, zrows, nbuf, obuf, abuf,
             gs0, gs1, gs2, gs3, ss0, ss1, ss2, ss3):
    core = lax.axis_index("c")
    sid = lax.axis_index("s")
    bufs = (rows, rows2, rows3, rows4)
    gsems = (gs0, gs1, gs2, gs3)
    ssems = (ss0, ss1, ss2, ss3)
    sem = gs0
    edges = (e0, e1, e2, e3)
    zs = (z0, z1, z2, z3)
    ns = (n0, n1, n2, n3)
    gus = (gu0, gu1, gu2, gu3)
    gis = (gi0, gi1, gi2, gi3)

    for gi in range(2):
        pltpu.sync_copy(zf2, rows)

        def zbody(p, carry):
            pltpu.sync_copy(rows, acc.at[pl.ds(sid * RPT + p * EC, EC)])
            return carry

        lax.fori_loop(0, RPT // EC, zbody, 0)
        plsc.subcore_barrier()

        for g in (gi, 2 + gi):
            @pl.when(core == g // 2)
            def _(g=g):
                def slab(t, carry):
                    base = sid * CPT + t * 16
                    pltpu.sync_copy(edges[g].at[0, pl.ds(base, 16)], srcb)
                    pltpu.sync_copy(edges[g].at[1, pl.ds(base, 16)], dstb)
                    for k in range(4):
                        pltpu.async_copy(
                            zs[g].at[srcb.at[k]], bufs[k], gsems[k])

                    def body(p, c2):
                        j = p * 4
                        for k in range(4):
                            pltpu.make_async_copy(
                                zs[g].at[srcb.at[j + k]], bufs[k],
                                gsems[k]).wait()
                            pltpu.async_copy(
                                bufs[k], acc.at[dstb.at[j + k]],
                                ssems[k], add=True)
                            if k >= 2:
                                _refill(p, j, k - 2)
                        _refill(p, j, 2)
                        _refill(p, j, 3)
                        return c2

                    def _refill(p, j, k):
                        pltpu.make_async_copy(
                            bufs[k], acc.at[dstb.at[j + k]], ssems[k]).wait()

                        @pl.when(p < 3)
                        def _():
                            pltpu.async_copy(
                                zs[g].at[srcb.at[j + 4 + k]], bufs[k],
                                gsems[k])

                    lax.fori_loop(0, 4, body, 0)
                    return carry

                lax.fori_loop(0, CPT // 16, slab, 0)

        plsc.subcore_barrier()

        for g in (gi, 2 + gi):
            @pl.when(core == g // 2)
            def _(g=g):
                for idx_src, out_ref in ((u_r, gus[g]), (i_r, gis[g])):
                    def rbody(kk, carry, idx_src=idx_src, out_ref=out_ref):
                        pltpu.sync_copy(idx_src.at[sid * IC + kk], idxb)
                        pltpu.async_copy(acc.at[idxb], grows, gs1)
                        pltpu.async_copy(zs[g].at[idxb], zrows, gs2)
                        pltpu.async_copy(ns[g].at[idxb], nbuf, gs3)
                        pltpu.make_async_copy(ns[g].at[idxb], nbuf, gs3).wait()

                        def rsq(q, c2):
                            sl = pl.ds(q * 16, 16)
                            v = nbuf[sl] + 1.0
                            iv = plsc.bitcast(v, jnp.int32)
                            iv = 0x5F3759DF - lax.shift_right_logical(iv, 1)
                            y = plsc.bitcast(iv, jnp.float32)
                            for _ in range(3):
                                y = y * (1.5 - 0.5 * v * y * y)
                            nbuf[sl] = y
                            return c2

                        lax.fori_loop(0, 8, rsq, 0)
                        pltpu.make_async_copy(acc.at[idxb], grows, gs1).wait()
                        pltpu.make_async_copy(
                            zs[g].at[idxb], zrows, gs2).wait()

                        def comp(q, c2):
                            nv16 = nbuf[pl.ds(q * 16, 16)]
                            for rr in range(16):
                                r = q * 16 + rr
                                nv = nv16[rr]
                                for c in range(D // 16):
                                    sl = pl.ds(c * 16, 16)
                                    v = (grows[r, sl] + zrows[r, sl]) * nv
                                    obuf[r, sl] = jnp.maximum(v, 0.0)
                            return c2

                        lax.fori_loop(0, 8, comp, 0)
                        pltpu.sync_copy(
                            obuf,
                            out_ref.at[pl.ds((sid * IC + kk) * 128, 128)])
                        return carry

                    lax.fori_loop(0, IC, rbody, 0)

        plsc.subcore_barrier()

    for cid, tab, out_ref in ((0, attn_a, oaa), (1, attn_b, obb)):
        @pl.when(core == cid)
        def _(tab=tab, out_ref=out_ref):
            def ab(kk, carry):
                pltpu.sync_copy(u_r.at[sid * IC + kk], idxb)
                pltpu.async_copy(tab.at[idxb], abuf, sem).wait()
                pltpu.sync_copy(
                    abuf, out_ref.at[pl.ds((sid * IC + kk) * 128, 128)])
                return carry

            lax.fori_loop(0, IC, ab, 0)


_mp_call = pl.kernel(
    _mp_body,
    out_type=([jax.ShapeDtypeStruct((BSZ, D), jnp.float32)] * 8
              + [jax.ShapeDtypeStruct((BSZ, F), jnp.float32)] * 2),
    mesh=_mesh,
    compiler_params=_sc_params,
    scratch_types=[
        pltpu.VMEM_SHARED((NPAD, D), jnp.float32),
        pltpu.VMEM((16, EC), jnp.int32),
        pltpu.VMEM((16, EC), jnp.int32),
        pltpu.VMEM((EC, D), jnp.float32),
        pltpu.VMEM((EC, D), jnp.float32),
        pltpu.VMEM((EC, D), jnp.float32),
        pltpu.VMEM((EC, D), jnp.float32),
        pltpu.VMEM((128,), jnp.int32),
        pltpu.VMEM((128, D), jnp.float32),
        pltpu.VMEM((128, D), jnp.float32),
        pltpu.VMEM((128,), jnp.float32),
        pltpu.VMEM((128, D), jnp.float32),
        pltpu.VMEM((128, F), jnp.float32),
    ] + [pltpu.SemaphoreType.DMA] * 8,
)


def _head_body(dom, g0u, g1u, g2u, g3u, g0i, g1i, g2i, g3i, waa, wbb,
               uw1, ub1, uw2, ub2, iw1, ib1, iw2, ib2,
               user_ref, item_ref):
    is_a = dom[0] == 0
    u_a = jnp.concatenate([g0u[...], g1u[...]], axis=1)
    u_b = jnp.concatenate([g2u[...], g3u[...]], axis=1)
    i_a = jnp.concatenate([g0i[...], g1i[...]], axis=1)
    i_b = jnp.concatenate([g2i[...], g3i[...]], axis=1)
    w_aa = waa[...]
    w_bb = wbb[...]
    u_from_a = u_a * w_aa + u_b * (1.0 - w_aa)
    u_from_b = u_b * w_bb + u_a * (1.0 - w_bb)
    user_in = jnp.where(is_a, u_from_a, u_from_b)
    item_in = jnp.where(is_a, i_a, i_b)

    def mlp(x, w1, b1, w2, b2):
        h = jnp.dot(x, w1[...], preferred_element_type=jnp.float32,
                    precision=lax.Precision.HIGHEST) + b1[...]
        h = jnp.maximum(h, 0.0)
        return jnp.dot(h, w2[...], preferred_element_type=jnp.float32,
                       precision=lax.Precision.HIGHEST) + b2[...]

    user_ref[...] = mlp(user_in, uw1, ub1, uw2, ub2)
    item_ref[...] = mlp(item_in, iw1, ib1, iw2, ib2)


_BLK = 512
_spec_d = pl.BlockSpec((_BLK, D), lambda b: (b, 0))
_spec_f = pl.BlockSpec((_BLK, F), lambda b: (b, 0))
_spec_g = pl.BlockSpec((NS * IC, 128), lambda b: (0, 0))
_spec_w = pl.BlockSpec((F, F), lambda b: (0, 0))
_spec_b = pl.BlockSpec((1, F), lambda b: (0, 0))

_head_call = pl.pallas_call(
    _head_body,
    grid=(BSZ // _BLK,),
    in_specs=([pl.BlockSpec(memory_space=pltpu.SMEM)]
              + [_spec_d] * 8 + [_spec_f] * 2
              + [_spec_w, _spec_b, _spec_w, _spec_b] * 2),
    out_specs=[_spec_f, _spec_f],
    out_shape=[jax.ShapeDtypeStruct((BSZ, F), jnp.float32)] * 2,
)


def kernel(u, i, domain, edge_rating_A, edge_review_A, edge_rating_B,
           edge_review_B, feat_rating_A, feat_review_A, feat_rating_B,
           feat_review_B, W_rat_A, W_rev_A, W_rat_B, W_rev_B, attn_AA,
           attn_BB, umlp_W1, umlp_b1, umlp_W2, umlp_b2, imlp_W1, imlp_b1,
           imlp_W2, imlp_b2):
    f32 = jnp.float32
    edges = (edge_rating_A, edge_review_A, edge_rating_B, edge_review_B)
    feats = (feat_rating_A, feat_review_A, feat_rating_B, feat_review_B)
    ws = (W_rat_A, W_rev_A, W_rat_B, W_rev_B)

    pad_j = jnp.arange(EPAD - E, dtype=jnp.int32)
    fill = jnp.stack([pad_j % N, N + pad_j % (NPAD - N)])
    edges_r = [jnp.concatenate([e.astype(jnp.int32), fill], axis=1)
               .reshape(2, NS * CPT, EC) for e in edges]
    u_r = u.astype(jnp.int32).reshape(NS * IC, 128)
    i_r = i.astype(jnp.int32).reshape(NS * IC, 128)

    zfill = jnp.zeros((RPT,), f32)
    ones_hbm = jnp.ones((EC,), f32)
    zf2 = jnp.zeros((EC, D), f32)

    degs = _deg_call(*edges_r, zfill, ones_hbm)

    zs = _z_call(*[feats[g].astype(f32) for g in range(4)],
                 *[ws[g].astype(f32) for g in range(4)],
                 *[degs[g].reshape(N, 1) for g in range(4)])

    mp_out = _mp_call(*edges_r, *zs, *degs, u_r, i_r,
                      attn_AA.astype(f32), attn_BB.astype(f32), zf2)
    gus = mp_out[0:4]
    gis = mp_out[4:8]
    oaa, obb = mp_out[8], mp_out[9]

    dom = jnp.asarray(domain, jnp.int32).reshape(1)
    user, item = _head_call(
        dom, gus[0], gus[1], gus[2], gus[3], gis[0], gis[1], gis[2], gis[3],
        oaa, obb, umlp_W1, umlp_b1.reshape(1, F), umlp_W2,
        umlp_b2.reshape(1, F), imlp_W1, imlp_b1.reshape(1, F), imlp_W2,
        imlp_b2.reshape(1, F))

    return (user, item, gus[1], gus[3])


# speedup vs baseline: 1.8350x; 1.0477x over previous
import functools

import jax
import jax.numpy as jnp
from jax import lax
from jax.experimental import pallas as pl
from jax.experimental.pallas import tpu as pltpu
from jax.experimental.pallas import tpu_sc as plsc

N = 10000
E = 320000
D = 64
F = 128
BSZ = 4096
NC, NS = 2, 16
EC = 128
CPT = 160
EPAD = NS * CPT * EC
NPAD = 10240
RPT = NPAD // NS
IC = BSZ // (NS * 128)

_mesh = plsc.VectorSubcoreMesh(core_axis_name="c", subcore_axis_name="s")


def _deg_body(d0, d1, d2, d3, zfill, ones_hbm,
              deg0, deg1, deg2, deg3,
              acc0, acc1, onesb, zb, dstb, tmpb, dsem):
    core = lax.axis_index("c")
    sid = lax.axis_index("s")
    edges = (d0, d1, d2, d3)
    degs = (deg0, deg1, deg2, deg3)
    accs = (acc0, acc1)

    pltpu.sync_copy(ones_hbm, onesb)
    pltpu.sync_copy(zfill, zb)
    for a in accs:
        pltpu.sync_copy(zb, a.at[pl.ds(sid * RPT, RPT)])

    plsc.subcore_barrier()

    for g in range(4):
        @pl.when(core == g // 2)
        def _(g=g):
            a = accs[g % 2]
            pltpu.sync_copy(edges[g].at[1, pl.ds(sid * CPT, CPT)], dstb)

            def body(p, carry):
                j = p * 8
                for k in range(8):
                    pltpu.async_copy(
                        onesb, a.at[dstb.at[j + k]], dsem, add=True)
                for k in range(8):
                    pltpu.make_async_copy(
                        onesb, a.at[dstb.at[j + k]], dsem).wait()
                return carry

            lax.fori_loop(0, CPT // 8, body, 0)

    plsc.subcore_barrier()

    for g in range(4):
        @pl.when((core == g // 2) & (sid < 10))
        def _(g=g):
            a = accs[g % 2]
            pltpu.sync_copy(a.at[pl.ds(sid * 1000, 1000)], tmpb)
            pltpu.sync_copy(tmpb, degs[g].at[pl.ds(sid * 1000, 1000)])


_sc_params = pltpu.CompilerParams(use_tc_tiling_on_sc=False,
                                  needs_layout_passes=False)

_deg_call = pl.kernel(
    _deg_body,
    out_type=[jax.ShapeDtypeStruct((N,), jnp.float32)] * 4,
    mesh=_mesh,
    compiler_params=_sc_params,
    scratch_types=[
        pltpu.VMEM_SHARED((NPAD,), jnp.float32),
        pltpu.VMEM_SHARED((NPAD,), jnp.float32),
        pltpu.VMEM((EC,), jnp.float32),
        pltpu.VMEM((RPT,), jnp.float32),
        pltpu.VMEM((CPT, EC), jnp.int32),
        pltpu.VMEM((1000,), jnp.float32),
        pltpu.SemaphoreType.DMA,
    ],
)


def _z_body(x0, x1, x2, x3, w0, w1, w2, w3, dg0, dg1, dg2, dg3,
            z0, z1, z2, z3):
    for x_ref, w_ref, deg_ref, z_ref in (
            (x0, w0, dg0, z0), (x1, w1, dg1, z1),
            (x2, w2, dg2, z2), (x3, w3, dg3, z3)):
        nrm = lax.rsqrt(deg_ref[...] + 1.0)
        y = jnp.dot(x_ref[...], w_ref[...],
                    preferred_element_type=jnp.float32,
                    precision=lax.Precision.DEFAULT)
        z_ref[...] = y * nrm


_z_call = pl.pallas_call(
    _z_body,
    grid=(10,),
    in_specs=([pl.BlockSpec((1000, F), lambda b: (b, 0))] * 4
              + [pl.BlockSpec((F, D), lambda b: (0, 0))] * 4
              + [pl.BlockSpec((1000, 1), lambda b: (b, 0))] * 4),
    out_specs=[pl.BlockSpec((1000, D), lambda b: (b, 0))] * 4,
    out_shape=[jax.ShapeDtypeStruct((N, D), jnp.float32)] * 4,
)


def _mp_body(e0, e1, e2, e3,
             z0, z1, z2, z3, n0, n1, n2, n3,
             u_r, i_r, attn_a, attn_b, zf2,
             gu0, gu1, gu2, gu3, gi0, gi1, gi2, gi3, oaa, obb,
             acc, srcb, dstb, rows, rows2, rows3, rows4,
             idxb, grows, zrows, nbuf, obuf, abuf,
             gs0, gs1, gs2, gs3, ss0, ss1, ss2, ss3):
    core = lax.axis_index("c")
    sid = lax.axis_index("s")
    bufs = (rows, rows2, rows3, rows4)
    gsems = (gs0, gs1, gs2, gs3)
    ssems = (ss0, ss1, ss2, ss3)
    sem = gs0
    edges = (e0, e1, e2, e3)
    zs = (z0, z1, z2, z3)
    ns = (n0, n1, n2, n3)
    gus = (gu0, gu1, gu2, gu3)
    gis = (gi0, gi1, gi2, gi3)

    for gi in range(2):
        pltpu.sync_copy(zf2, rows)

        def zbody(p, carry):
            pltpu.sync_copy(rows, acc.at[pl.ds(sid * RPT + p * EC, EC)])
            return carry

        lax.fori_loop(0, RPT // EC, zbody, 0)
        plsc.subcore_barrier()

        for g in (gi, 2 + gi):
            @pl.when(core == g // 2)
            def _(g=g):
                def slab(t, carry):
                    base = sid * CPT + t * 16
                    pltpu.sync_copy(edges[g].at[0, pl.ds(base, 16)], srcb)
                    pltpu.sync_copy(edges[g].at[1, pl.ds(base, 16)], dstb)
                    for k in range(4):
                        pltpu.async_copy(
                            zs[g].at[srcb.at[k]], bufs[k], gsems[k])

                    def body(p, c2):
                        j = p * 4
                        for k in range(4):
                            pltpu.make_async_copy(
                                zs[g].at[srcb.at[j + k]], bufs[k],
                                gsems[k]).wait()
                            pltpu.async_copy(
                                bufs[k], acc.at[dstb.at[j + k]],
                                ssems[k], add=True)
                            if k >= 2:
                                _refill(p, j, k - 2)
                        _refill(p, j, 2)
                        _refill(p, j, 3)
                        return c2

                    def _refill(p, j, k):
                        pltpu.make_async_copy(
                            bufs[k], acc.at[dstb.at[j + k]], ssems[k]).wait()

                        @pl.when(p < 3)
                        def _():
                            pltpu.async_copy(
                                zs[g].at[srcb.at[j + 4 + k]], bufs[k],
                                gsems[k])

                    lax.fori_loop(0, 4, body, 0)
                    return carry

                lax.fori_loop(0, CPT // 16, slab, 0)

        plsc.subcore_barrier()

        for g in (gi, 2 + gi):
            @pl.when(core == g // 2)
            def _(g=g):
                for idx_src, out_ref in ((u_r, gus[g]), (i_r, gis[g])):
                    def rbody(kk, carry, idx_src=idx_src, out_ref=out_ref):
                        pltpu.sync_copy(idx_src.at[sid * IC + kk], idxb)
                        pltpu.async_copy(acc.at[idxb], grows, gs1)
                        pltpu.async_copy(zs[g].at[idxb], zrows, gs2)
                        pltpu.async_copy(ns[g].at[idxb], nbuf, gs3)
                        pltpu.make_async_copy(ns[g].at[idxb], nbuf, gs3).wait()

                        def rsq(q, c2):
                            sl = pl.ds(q * 16, 16)
                            v = nbuf[sl] + 1.0
                            iv = plsc.bitcast(v, jnp.int32)
                            iv = 0x5F3759DF - lax.shift_right_logical(iv, 1)
                            y = plsc.bitcast(iv, jnp.float32)
                            for _ in range(3):
                                y = y * (1.5 - 0.5 * v * y * y)
                            nbuf[sl] = y
                            return c2

                        lax.fori_loop(0, 8, rsq, 0)
                        pltpu.make_async_copy(acc.at[idxb], grows, gs1).wait()
                        pltpu.make_async_copy(
                            zs[g].at[idxb], zrows, gs2).wait()

                        def comp(q, c2):
                            nv16 = nbuf[pl.ds(q * 16, 16)]
                            for rr in range(16):
                                r = q * 16 + rr
                                nv = nv16[rr]
                                for c in range(D // 16):
                                    sl = pl.ds(c * 16, 16)
                                    v = (grows[r, sl] + zrows[r, sl]) * nv
                                    obuf[r, sl] = jnp.maximum(v, 0.0)
                            return c2

                        lax.fori_loop(0, 8, comp, 0)
                        pltpu.sync_copy(
                            obuf,
                            out_ref.at[pl.ds((sid * IC + kk) * 128, 128)])
                        return carry

                    lax.fori_loop(0, IC, rbody, 0)

        plsc.subcore_barrier()

    for cid, tab, out_ref in ((0, attn_a, oaa), (1, attn_b, obb)):
        @pl.when(core == cid)
        def _(tab=tab, out_ref=out_ref):
            def ab(kk, carry):
                pltpu.sync_copy(u_r.at[sid * IC + kk], idxb)
                pltpu.async_copy(tab.at[idxb], abuf, sem).wait()
                pltpu.sync_copy(
                    abuf, out_ref.at[pl.ds((sid * IC + kk) * 128, 128)])
                return carry

            lax.fori_loop(0, IC, ab, 0)


_mp_call = pl.kernel(
    _mp_body,
    out_type=([jax.ShapeDtypeStruct((BSZ, D), jnp.float32)] * 8
              + [jax.ShapeDtypeStruct((BSZ, F), jnp.float32)] * 2),
    mesh=_mesh,
    compiler_params=_sc_params,
    scratch_types=[
        pltpu.VMEM_SHARED((NPAD, D), jnp.float32),
        pltpu.VMEM((16, EC), jnp.int32),
        pltpu.VMEM((16, EC), jnp.int32),
        pltpu.VMEM((EC, D), jnp.float32),
        pltpu.VMEM((EC, D), jnp.float32),
        pltpu.VMEM((EC, D), jnp.float32),
        pltpu.VMEM((EC, D), jnp.float32),
        pltpu.VMEM((128,), jnp.int32),
        pltpu.VMEM((128, D), jnp.float32),
        pltpu.VMEM((128, D), jnp.float32),
        pltpu.VMEM((128,), jnp.float32),
        pltpu.VMEM((128, D), jnp.float32),
        pltpu.VMEM((128, F), jnp.float32),
    ] + [pltpu.SemaphoreType.DMA] * 8,
)


def _head_body(dom, g0u, g1u, g2u, g3u, g0i, g1i, g2i, g3i, waa, wbb,
               uw1, ub1, uw2, ub2, iw1, ib1, iw2, ib2,
               user_ref, item_ref):
    is_a = dom[0] == 0
    u_a = jnp.concatenate([g0u[...], g1u[...]], axis=1)
    u_b = jnp.concatenate([g2u[...], g3u[...]], axis=1)
    i_a = jnp.concatenate([g0i[...], g1i[...]], axis=1)
    i_b = jnp.concatenate([g2i[...], g3i[...]], axis=1)
    w_aa = waa[...]
    w_bb = wbb[...]
    u_from_a = u_a * w_aa + u_b * (1.0 - w_aa)
    u_from_b = u_b * w_bb + u_a * (1.0 - w_bb)
    user_in = jnp.where(is_a, u_from_a, u_from_b)
    item_in = jnp.where(is_a, i_a, i_b)

    def mlp(x, w1, b1, w2, b2):
        h = jnp.dot(x, w1[...], preferred_element_type=jnp.float32,
                    precision=lax.Precision.DEFAULT) + b1[...]
        h = jnp.maximum(h, 0.0)
        return jnp.dot(h, w2[...], preferred_element_type=jnp.float32,
                       precision=lax.Precision.DEFAULT) + b2[...]

    user_ref[...] = mlp(user_in, uw1, ub1, uw2, ub2)
    item_ref[...] = mlp(item_in, iw1, ib1, iw2, ib2)


_BLK = 512
_spec_d = pl.BlockSpec((_BLK, D), lambda b: (b, 0))
_spec_f = pl.BlockSpec((_BLK, F), lambda b: (b, 0))
_spec_g = pl.BlockSpec((NS * IC, 128), lambda b: (0, 0))
_spec_w = pl.BlockSpec((F, F), lambda b: (0, 0))
_spec_b = pl.BlockSpec((1, F), lambda b: (0, 0))

_head_call = pl.pallas_call(
    _head_body,
    grid=(BSZ // _BLK,),
    in_specs=([pl.BlockSpec(memory_space=pltpu.SMEM)]
              + [_spec_d] * 8 + [_spec_f] * 2
              + [_spec_w, _spec_b, _spec_w, _spec_b] * 2),
    out_specs=[_spec_f, _spec_f],
    out_shape=[jax.ShapeDtypeStruct((BSZ, F), jnp.float32)] * 2,
)


def kernel(u, i, domain, edge_rating_A, edge_review_A, edge_rating_B,
           edge_review_B, feat_rating_A, feat_review_A, feat_rating_B,
           feat_review_B, W_rat_A, W_rev_A, W_rat_B, W_rev_B, attn_AA,
           attn_BB, umlp_W1, umlp_b1, umlp_W2, umlp_b2, imlp_W1, imlp_b1,
           imlp_W2, imlp_b2):
    f32 = jnp.float32
    edges = (edge_rating_A, edge_review_A, edge_rating_B, edge_review_B)
    feats = (feat_rating_A, feat_review_A, feat_rating_B, feat_review_B)
    ws = (W_rat_A, W_rev_A, W_rat_B, W_rev_B)

    pad_j = jnp.arange(EPAD - E, dtype=jnp.int32)
    fill = jnp.stack([pad_j % N, N + pad_j % (NPAD - N)])
    edges_r = [jnp.concatenate([e.astype(jnp.int32), fill], axis=1)
               .reshape(2, NS * CPT, EC) for e in edges]
    u_r = u.astype(jnp.int32).reshape(NS * IC, 128)
    i_r = i.astype(jnp.int32).reshape(NS * IC, 128)

    zfill = jnp.zeros((RPT,), f32)
    ones_hbm = jnp.ones((EC,), f32)
    zf2 = jnp.zeros((EC, D), f32)

    degs = _deg_call(*edges_r, zfill, ones_hbm)

    zs = _z_call(*[feats[g].astype(f32) for g in range(4)],
                 *[ws[g].astype(f32) for g in range(4)],
                 *[degs[g].reshape(N, 1) for g in range(4)])

    mp_out = _mp_call(*edges_r, *zs, *degs, u_r, i_r,
                      attn_AA.astype(f32), attn_BB.astype(f32), zf2)
    gus = mp_out[0:4]
    gis = mp_out[4:8]
    oaa, obb = mp_out[8], mp_out[9]

    dom = jnp.asarray(domain, jnp.int32).reshape(1)
    user, item = _head_call(
        dom, gus[0], gus[1], gus[2], gus[3], gis[0], gis[1], gis[2], gis[3],
        oaa, obb, umlp_W1, umlp_b1.reshape(1, F), umlp_W2,
        umlp_b2.reshape(1, F), imlp_W1, imlp_b1.reshape(1, F), imlp_W2,
        imlp_b2.reshape(1, F))

    return (user, item, gus[1], gus[3])


# speedup vs baseline: 1.8356x; 1.0003x over previous
import jax
import jax.numpy as jnp
from jax import lax
from jax.experimental import pallas as pl
from jax.experimental.pallas import tpu as pltpu
from jax.experimental.pallas import tpu_sc as plsc

N = 10000
E = 320000
D = 64
F = 128
BSZ = 4096
NC, NS = 2, 16
EC = 128
CPT = 160
EPAD = NS * CPT * EC
NPAD = 10240
RPT = NPAD // NS
IC = BSZ // (NS * 128)

_mesh = plsc.VectorSubcoreMesh(core_axis_name="c", subcore_axis_name="s")


def _deg_body(d0, d1, d2, d3, zfill, ones_hbm,
              deg0, deg1, deg2, deg3,
              acc0, acc1, onesb, zb, dstb, tmpb, dsem):
    core = lax.axis_index("c")
    sid = lax.axis_index("s")
    edges = (d0, d1, d2, d3)
    degs = (deg0, deg1, deg2, deg3)
    accs = (acc0, acc1)

    pltpu.sync_copy(ones_hbm, onesb)
    pltpu.sync_copy(zfill, zb)
    for a in accs:
        pltpu.sync_copy(zb, a.at[pl.ds(sid * RPT, RPT)])

    plsc.subcore_barrier()

    for g in range(4):
        @pl.when(core == g // 2)
        def _(g=g):
            a = accs[g % 2]
            pltpu.sync_copy(edges[g].at[1, pl.ds(sid * CPT, CPT)], dstb)

            def body(p, carry):
                j = p * 8
                for k in range(8):
                    pltpu.async_copy(
                        onesb, a.at[dstb.at[j + k]], dsem, add=True)
                for k in range(8):
                    pltpu.make_async_copy(
                        onesb, a.at[dstb.at[j + k]], dsem).wait()
                return carry

            lax.fori_loop(0, CPT // 8, body, 0)

    plsc.subcore_barrier()

    for g in range(4):
        @pl.when((core == g // 2) & (sid < 10))
        def _(g=g):
            a = accs[g % 2]
            pltpu.sync_copy(a.at[pl.ds(sid * 1000, 1000)], tmpb)
            pltpu.sync_copy(tmpb, degs[g].at[pl.ds(sid * 1000, 1000)])


_sc_params = pltpu.CompilerParams(use_tc_tiling_on_sc=False,
                                  needs_layout_passes=False)

_deg_call = pl.kernel(
    _deg_body,
    out_type=[jax.ShapeDtypeStruct((N,), jnp.float32)] * 4,
    mesh=_mesh,
    compiler_params=_sc_params,
    scratch_types=[
        pltpu.VMEM_SHARED((NPAD,), jnp.float32),
        pltpu.VMEM_SHARED((NPAD,), jnp.float32),
        pltpu.VMEM((EC,), jnp.float32),
        pltpu.VMEM((RPT,), jnp.float32),
        pltpu.VMEM((CPT, EC), jnp.int32),
        pltpu.VMEM((1000,), jnp.float32),
        pltpu.SemaphoreType.DMA,
    ],
)


def _z_body(x0, x1, x2, x3, w0, w1, w2, w3, dg0, dg1, dg2, dg3,
            z0, z1, z2, z3):
    for x_ref, w_ref, deg_ref, z_ref in (
            (x0, w0, dg0, z0), (x1, w1, dg1, z1),
            (x2, w2, dg2, z2), (x3, w3, dg3, z3)):
        nrm = lax.rsqrt(deg_ref[...] + 1.0)
        y = jnp.dot(x_ref[...], w_ref[...],
                    preferred_element_type=jnp.float32,
                    precision=lax.Precision.DEFAULT)
        z_ref[...] = y * nrm


_z_call = pl.pallas_call(
    _z_body,
    grid=(10,),
    in_specs=([pl.BlockSpec((1000, F), lambda b: (b, 0))] * 4
              + [pl.BlockSpec((F, D), lambda b: (0, 0))] * 4
              + [pl.BlockSpec((1000, 1), lambda b: (b, 0))] * 4),
    out_specs=[pl.BlockSpec((1000, D), lambda b: (b, 0))] * 4,
    out_shape=[jax.ShapeDtypeStruct((N, D), jnp.float32)] * 4,
)


def _mp_body(e0, e1, e2, e3,
             z0, z1, z2, z3, n0, n1, n2, n3,
             u_r, i_r, attn_a, attn_b, zf2,
             gu0, gu1, gu2, gu3, gi0, gi1, gi2, gi3, oaa, obb,
             acc, srcb, dstb, rows, rows2, rows3, rows4,
             idxb, grows, zrows, nbuf, obuf, abuf,
             gs0, gs1, gs2, gs3, ss0, ss1, ss2, ss3):
    core = lax.axis_index("c")
    sid = lax.axis_index("s")
    bufs = (rows, rows2, rows3, rows4)
    gsems = (gs0, gs1, gs2, gs3)
    ssems = (ss0, ss1, ss2, ss3)
    sem = gs0
    edges = (e0, e1, e2, e3)
    zs = (z0, z1, z2, z3)
    ns = (n0, n1, n2, n3)
    gus = (gu0, gu1, gu2, gu3)
    gis = (gi0, gi1, gi2, gi3)

    for gi in range(2):
        pltpu.sync_copy(zf2, rows)

        def zbody(p, carry):
            pltpu.sync_copy(rows, acc.at[pl.ds(sid * RPT + p * EC, EC)])
            return carry

        lax.fori_loop(0, RPT // EC, zbody, 0)
        plsc.subcore_barrier()

        for g in (gi, 2 + gi):
            @pl.when(core == g // 2)
            def _(g=g):
                def slab(t, carry):
                    base = sid * CPT + t * 16
                    pltpu.sync_copy(edges[g].at[0, pl.ds(base, 16)], srcb)
                    pltpu.sync_copy(edges[g].at[1, pl.ds(base, 16)], dstb)
                    for k in range(4):
                        pltpu.async_copy(
                            zs[g].at[srcb.at[k]], bufs[k], gsems[k])

                    def body(p, c2):
                        j = p * 4
                        for k in range(4):
                            pltpu.make_async_copy(
                                zs[g].at[srcb.at[j + k]], bufs[k],
                                gsems[k]).wait()
                            pltpu.async_copy(
                                bufs[k], acc.at[dstb.at[j + k]],
                                ssems[k], add=True)
                            if k >= 2:
                                _refill(p, j, k - 2)
                        _refill(p, j, 2)
                        _refill(p, j, 3)
                        return c2

                    def _refill(p, j, k):
                        pltpu.make_async_copy(
                            bufs[k], acc.at[dstb.at[j + k]], ssems[k]).wait()

                        @pl.when(p < 3)
                        def _():
                            pltpu.async_copy(
                                zs[g].at[srcb.at[j + 4 + k]], bufs[k],
                                gsems[k])

                    lax.fori_loop(0, 4, body, 0)
                    return carry

                lax.fori_loop(0, CPT // 16, slab, 0)

        plsc.subcore_barrier()

        for g in (gi, 2 + gi):
            @pl.when(core == g // 2)
            def _(g=g):
                for idx_src, out_ref in ((u_r, gus[g]), (i_r, gis[g])):
                    def rbody(kk, carry, idx_src=idx_src, out_ref=out_ref):
                        pltpu.sync_copy(idx_src.at[sid * IC + kk], idxb)
                        pltpu.async_copy(acc.at[idxb], grows, gs1)
                        pltpu.async_copy(zs[g].at[idxb], zrows, gs2)
                        pltpu.async_copy(ns[g].at[idxb], nbuf, gs3)
                        pltpu.make_async_copy(ns[g].at[idxb], nbuf, gs3).wait()

                        def rsq(q, c2):
                            sl = pl.ds(q * 16, 16)
                            v = nbuf[sl] + 1.0
                            iv = plsc.bitcast(v, jnp.int32)
                            iv = 0x5F3759DF - lax.shift_right_logical(iv, 1)
                            y = plsc.bitcast(iv, jnp.float32)
                            for _ in range(3):
                                y = y * (1.5 - 0.5 * v * y * y)
                            nbuf[sl] = y
                            return c2

                        lax.fori_loop(0, 8, rsq, 0)
                        pltpu.make_async_copy(acc.at[idxb], grows, gs1).wait()
                        pltpu.make_async_copy(
                            zs[g].at[idxb], zrows, gs2).wait()

                        def comp(q, c2):
                            nv16 = nbuf[pl.ds(q * 16, 16)]
                            for rr in range(16):
                                r = q * 16 + rr
                                nv = nv16[rr]
                                for c in range(D // 16):
                                    sl = pl.ds(c * 16, 16)
                                    v = (grows[r, sl] + zrows[r, sl]) * nv
                                    obuf[r, sl] = jnp.maximum(v, 0.0)
                            return c2

                        lax.fori_loop(0, 8, comp, 0)
                        pltpu.sync_copy(
                            obuf,
                            out_ref.at[pl.ds((sid * IC + kk) * 128, 128)])
                        return carry

                    lax.fori_loop(0, IC, rbody, 0)

        plsc.subcore_barrier()

    for cid, tab, out_ref in ((0, attn_a, oaa), (1, attn_b, obb)):
        @pl.when(core == cid)
        def _(tab=tab, out_ref=out_ref):
            def ab(kk, carry):
                pltpu.sync_copy(u_r.at[sid * IC + kk], idxb)
                pltpu.async_copy(tab.at[idxb], abuf, sem).wait()
                pltpu.sync_copy(
                    abuf, out_ref.at[pl.ds((sid * IC + kk) * 128, 128)])
                return carry

            lax.fori_loop(0, IC, ab, 0)


_mp_call = pl.kernel(
    _mp_body,
    out_type=([jax.ShapeDtypeStruct((BSZ, D), jnp.float32)] * 8
              + [jax.ShapeDtypeStruct((BSZ, F), jnp.float32)] * 2),
    mesh=_mesh,
    compiler_params=_sc_params,
    scratch_types=[
        pltpu.VMEM_SHARED((NPAD, D), jnp.float32),
        pltpu.VMEM((16, EC), jnp.int32),
        pltpu.VMEM((16, EC), jnp.int32),
        pltpu.VMEM((EC, D), jnp.float32),
        pltpu.VMEM((EC, D), jnp.float32),
        pltpu.VMEM((EC, D), jnp.float32),
        pltpu.VMEM((EC, D), jnp.float32),
        pltpu.VMEM((128,), jnp.int32),
        pltpu.VMEM((128, D), jnp.float32),
        pltpu.VMEM((128, D), jnp.float32),
        pltpu.VMEM((128,), jnp.float32),
        pltpu.VMEM((128, D), jnp.float32),
        pltpu.VMEM((128, F), jnp.float32),
    ] + [pltpu.SemaphoreType.DMA] * 8,
)


def _head_body(dom, g0u, g1u, g2u, g3u, g0i, g1i, g2i, g3i, waa, wbb,
               uw1, ub1, uw2, ub2, iw1, ib1, iw2, ib2,
               user_ref, item_ref):
    is_a = dom[0] == 0
    u_a = jnp.concatenate([g0u[...], g1u[...]], axis=1)
    u_b = jnp.concatenate([g2u[...], g3u[...]], axis=1)
    i_a = jnp.concatenate([g0i[...], g1i[...]], axis=1)
    i_b = jnp.concatenate([g2i[...], g3i[...]], axis=1)
    w_aa = waa[...]
    w_bb = wbb[...]
    u_from_a = u_a * w_aa + u_b * (1.0 - w_aa)
    u_from_b = u_b * w_bb + u_a * (1.0 - w_bb)
    user_in = jnp.where(is_a, u_from_a, u_from_b)
    item_in = jnp.where(is_a, i_a, i_b)

    def mlp(x, w1, b1, w2, b2):
        h = jnp.dot(x, w1[...], preferred_element_type=jnp.float32,
                    precision=lax.Precision.DEFAULT) + b1[...]
        h = jnp.maximum(h, 0.0)
        return jnp.dot(h, w2[...], preferred_element_type=jnp.float32,
                       precision=lax.Precision.DEFAULT) + b2[...]

    user_ref[...] = mlp(user_in, uw1, ub1, uw2, ub2)
    item_ref[...] = mlp(item_in, iw1, ib1, iw2, ib2)


_BLK = 512
_spec_d = pl.BlockSpec((_BLK, D), lambda b: (b, 0))
_spec_f = pl.BlockSpec((_BLK, F), lambda b: (b, 0))
_spec_w = pl.BlockSpec((F, F), lambda b: (0, 0))
_spec_b = pl.BlockSpec((1, F), lambda b: (0, 0))

_head_call = pl.pallas_call(
    _head_body,
    grid=(BSZ // _BLK,),
    in_specs=([pl.BlockSpec(memory_space=pltpu.SMEM)]
              + [_spec_d] * 8 + [_spec_f] * 2
              + [_spec_w, _spec_b, _spec_w, _spec_b] * 2),
    out_specs=[_spec_f, _spec_f],
    out_shape=[jax.ShapeDtypeStruct((BSZ, F), jnp.float32)] * 2,
)


def kernel(u, i, domain, edge_rating_A, edge_review_A, edge_rating_B,
           edge_review_B, feat_rating_A, feat_review_A, feat_rating_B,
           feat_review_B, W_rat_A, W_rev_A, W_rat_B, W_rev_B, attn_AA,
           attn_BB, umlp_W1, umlp_b1, umlp_W2, umlp_b2, imlp_W1, imlp_b1,
           imlp_W2, imlp_b2):
    f32 = jnp.float32
    edges = (edge_rating_A, edge_review_A, edge_rating_B, edge_review_B)
    feats = (feat_rating_A, feat_review_A, feat_rating_B, feat_review_B)
    ws = (W_rat_A, W_rev_A, W_rat_B, W_rev_B)

    pad_j = jnp.arange(EPAD - E, dtype=jnp.int32)
    fill = jnp.stack([pad_j % N, N + pad_j % (NPAD - N)])
    edges_r = [jnp.concatenate([e.astype(jnp.int32), fill], axis=1)
               .reshape(2, NS * CPT, EC) for e in edges]
    u_r = u.astype(jnp.int32).reshape(NS * IC, 128)
    i_r = i.astype(jnp.int32).reshape(NS * IC, 128)

    zfill = jnp.zeros((RPT,), f32)
    ones_hbm = jnp.ones((EC,), f32)
    zf2 = jnp.zeros((EC, D), f32)

    degs = _deg_call(*edges_r, zfill, ones_hbm)

    zs = _z_call(*[feats[g].astype(f32) for g in range(4)],
                 *[ws[g].astype(f32) for g in range(4)],
                 *[degs[g].reshape(N, 1) for g in range(4)])

    mp_out = _mp_call(*edges_r, *zs, *degs, u_r, i_r,
                      attn_AA.astype(f32), attn_BB.astype(f32), zf2)
    gus = mp_out[0:4]
    gis = mp_out[4:8]
    oaa, obb = mp_out[8], mp_out[9]

    dom = jnp.asarray(domain, jnp.int32).reshape(1)
    user, item = _head_call(
        dom, gus[0], gus[1], gus[2], gus[3], gis[0], gis[1], gis[2], gis[3],
        oaa, obb, umlp_W1, umlp_b1.reshape(1, F), umlp_W2,
        umlp_b2.reshape(1, F), imlp_W1, imlp_b1.reshape(1, F), imlp_W2,
        imlp_b2.reshape(1, F))

    return (user, item, gus[1], gus[3])
